# baseline (speedup 1.0000x reference)
import jax
import jax.numpy as jnp
from jax import lax
from jax.experimental import pallas as pl
from jax.experimental.pallas import tpu as pltpu
from jax.experimental.pallas import tpu_sc as plsc

NC = 2
NS = 16
L = 16
NW = NC * NS
B = 16384
D = 128
DC = D // L
BPW = B // NW
CHUNK = 128
NCHUNK = BPW // CHUNK
EPS = 1e-6
GAMMA = 1.0


def _rsqrt(x):
    i = lax.bitcast_convert_type(x, jnp.int32)
    i = jnp.int32(0x5F3759DF) - lax.shift_right_logical(i, 1)
    y = lax.bitcast_convert_type(i, jnp.float32)
    for _ in range(3):
        y = y * (1.5 - 0.5 * x * y * y)
    return y


def _tec_body(head_r, rel_r, tail_r, nh_r, etab, rtab, out_r,
              idx_h, idx_r, idx_t, idx_n, rh, rr, rt, rn, loss_buf, sem):
    cid = lax.axis_index("c")
    sid = lax.axis_index("s")
    wid = sid * NC + cid
    base = wid * BPW

    pltpu.sync_copy(head_r.at[pl.ds(base, BPW)], idx_h)
    pltpu.sync_copy(rel_r.at[pl.ds(base, BPW)], idx_r)
    pltpu.sync_copy(tail_r.at[pl.ds(base, BPW)], idx_t)
    pltpu.sync_copy(nh_r.at[pl.ds(base, BPW)], idx_n)

    loss = jnp.float32(0.0)

    for ck in range(NCHUNK):
        o = ck * CHUNK
        cps = [
            pltpu.async_copy(etab.at[idx_h.at[pl.ds(o, CHUNK)]], rh, sem),
            pltpu.async_copy(rtab.at[idx_r.at[pl.ds(o, CHUNK)]], rr, sem),
            pltpu.async_copy(etab.at[idx_t.at[pl.ds(o, CHUNK)]], rt, sem),
            pltpu.async_copy(etab.at[idx_n.at[pl.ds(o, CHUNK)]], rn, sem),
        ]
        for cp in cps:
            cp.wait()

        @plsc.parallel_loop(0, CHUNK, unroll=4, carry=loss)
        def elem_body(e, loss):
            lh = [rh[e, pl.ds(k * L, L)] for k in range(DC)]
            lr = [rr[e, pl.ds(k * L, L)] for k in range(DC)]
            lt = [rt[e, pl.ds(k * L, L)] for k in range(DC)]
            ln = [rn[e, pl.ds(k * L, L)] for k in range(DC)]

            def sumsq(vs):
                acc = vs[0] * vs[0]
                for v in vs[1:]:
                    acc = acc + v * v
                return jnp.sum(acc)

            ah = _rsqrt(jnp.maximum(sumsq(lh), 1e-24))
            ar = _rsqrt(jnp.maximum(sumsq(lr), 1e-24))
            at = _rsqrt(jnp.maximum(sumsq(lt), 1e-24))
            an = _rsqrt(jnp.maximum(sumsq(ln), 1e-24))

            acc1 = None
            acc2 = None
            for k in range(DC):
                w = lr[k] * ar - lt[k] * at + EPS
                u = lh[k] * ah + w
                v = ln[k] * an + w
                acc1 = u * u if acc1 is None else acc1 + u * u
                acc2 = v * v if acc2 is None else acc2 + v * v
            s1 = jnp.maximum(jnp.sum(acc1), 1e-30)
            s2 = jnp.maximum(jnp.sum(acc2), 1e-30)
            d1 = s1 * _rsqrt(s1)
            d2 = s2 * _rsqrt(s2)
            return loss + jnp.maximum(GAMMA + d1 - d2, 0.0)

        loss = elem_body

    lane = lax.broadcasted_iota(jnp.int32, (L,), 0)
    loss_buf[...] = jnp.where(lane == 0, loss, 0.0)
    pltpu.sync_copy(loss_buf, out_r.at[wid])


@jax.jit
def _transe_loss_partials(head, rel, tail, negative_head, entity_table,
                          relation_table):
    mesh = plsc.VectorSubcoreMesh(
        core_axis_name="c", subcore_axis_name="s", num_cores=NC,
        num_subcores=NS)
    f = pl.kernel(
        _tec_body,
        out_type=jax.ShapeDtypeStruct((NW, L), jnp.float32),
        mesh=mesh,
        compiler_params=pltpu.CompilerParams(needs_layout_passes=False),
        scratch_types=[
            pltpu.VMEM((BPW,), jnp.int32),
            pltpu.VMEM((BPW,), jnp.int32),
            pltpu.VMEM((BPW,), jnp.int32),
            pltpu.VMEM((BPW,), jnp.int32),
            pltpu.VMEM((CHUNK, D), jnp.float32),
            pltpu.VMEM((CHUNK, D), jnp.float32),
            pltpu.VMEM((CHUNK, D), jnp.float32),
            pltpu.VMEM((CHUNK, D), jnp.float32),
            pltpu.VMEM((L,), jnp.float32),
            pltpu.SemaphoreType.DMA,
        ],
    )
    return f(head, rel, tail, negative_head, entity_table, relation_table)


def kernel(head, rel, tail, negative_head, negative_tail, entity_table,
           relation_table):
    del negative_tail
    partials = _transe_loss_partials(head, rel, tail, negative_head,
                                     entity_table, relation_table)
    return jnp.sum(partials)


# speedup vs baseline: 1.1647x; 1.1647x over previous
import jax
import jax.numpy as jnp
from jax import lax
from jax.experimental import pallas as pl
from jax.experimental.pallas import tpu as pltpu
from jax.experimental.pallas import tpu_sc as plsc

NC = 2
NS = 16
L = 16
NW = NC * NS
B = 16384
D = 128
DC = D // L
BPW = B // NW
CHUNK = 128
NCHUNK = BPW // CHUNK
EPS = 1e-6
GAMMA = 1.0


def _rsqrt(x):
    i = lax.bitcast_convert_type(x, jnp.int32)
    i = jnp.int32(0x5F3759DF) - lax.shift_right_logical(i, 1)
    y = lax.bitcast_convert_type(i, jnp.float32)
    for _ in range(3):
        y = y * (1.5 - 0.5 * x * y * y)
    return y


def _tec_body(head_r, rel_r, tail_r, nh_r, etab, rtab, out_r,
              idx_h, idx_r, idx_t, idx_n, rh, rr, rt, rn, loss_buf, sem):
    cid = lax.axis_index("c")
    sid = lax.axis_index("s")
    wid = sid * NC + cid
    base = wid * BPW

    pltpu.sync_copy(head_r.at[pl.ds(base, BPW)], idx_h)
    pltpu.sync_copy(rel_r.at[pl.ds(base, BPW)], idx_r)
    pltpu.sync_copy(tail_r.at[pl.ds(base, BPW)], idx_t)
    pltpu.sync_copy(nh_r.at[pl.ds(base, BPW)], idx_n)

    loss = jnp.float32(0.0)

    for ck in range(NCHUNK):
        o = ck * CHUNK
        cps = [
            pltpu.async_copy(etab.at[idx_h.at[pl.ds(o, CHUNK)]], rh, sem),
            pltpu.async_copy(rtab.at[idx_r.at[pl.ds(o, CHUNK)]], rr, sem),
            pltpu.async_copy(etab.at[idx_t.at[pl.ds(o, CHUNK)]], rt, sem),
            pltpu.async_copy(etab.at[idx_n.at[pl.ds(o, CHUNK)]], rn, sem),
        ]
        for cp in cps:
            cp.wait()

        @plsc.parallel_loop(0, CHUNK, unroll=1, carry=loss)
        def elem_body(e, loss):
            lh = [rh[e, pl.ds(k * L, L)] for k in range(DC)]
            lr = [rr[e, pl.ds(k * L, L)] for k in range(DC)]
            lt = [rt[e, pl.ds(k * L, L)] for k in range(DC)]
            ln = [rn[e, pl.ds(k * L, L)] for k in range(DC)]

            def sumsq(vs):
                acc = vs[0] * vs[0]
                for v in vs[1:]:
                    acc = acc + v * v
                return jnp.sum(acc)

            ah = _rsqrt(jnp.maximum(sumsq(lh), 1e-24))
            ar = _rsqrt(jnp.maximum(sumsq(lr), 1e-24))
            at = _rsqrt(jnp.maximum(sumsq(lt), 1e-24))
            an = _rsqrt(jnp.maximum(sumsq(ln), 1e-24))

            acc1 = None
            acc2 = None
            for k in range(DC):
                w = lr[k] * ar - lt[k] * at + EPS
                u = lh[k] * ah + w
                v = ln[k] * an + w
                acc1 = u * u if acc1 is None else acc1 + u * u
                acc2 = v * v if acc2 is None else acc2 + v * v
            s1 = jnp.maximum(jnp.sum(acc1), 1e-30)
            s2 = jnp.maximum(jnp.sum(acc2), 1e-30)
            d1 = s1 * _rsqrt(s1)
            d2 = s2 * _rsqrt(s2)
            return loss + jnp.maximum(GAMMA + d1 - d2, 0.0)

        loss = elem_body

    lane = lax.broadcasted_iota(jnp.int32, (L,), 0)
    loss_buf[...] = jnp.where(lane == 0, loss, 0.0)
    pltpu.sync_copy(loss_buf, out_r.at[wid])


@jax.jit
def _transe_loss_partials(head, rel, tail, negative_head, entity_table,
                          relation_table):
    mesh = plsc.VectorSubcoreMesh(
        core_axis_name="c", subcore_axis_name="s", num_cores=NC,
        num_subcores=NS)
    f = pl.kernel(
        _tec_body,
        out_type=jax.ShapeDtypeStruct((NW, L), jnp.float32),
        mesh=mesh,
        compiler_params=pltpu.CompilerParams(needs_layout_passes=False),
        scratch_types=[
            pltpu.VMEM((BPW,), jnp.int32),
            pltpu.VMEM((BPW,), jnp.int32),
            pltpu.VMEM((BPW,), jnp.int32),
            pltpu.VMEM((BPW,), jnp.int32),
            pltpu.VMEM((CHUNK, D), jnp.float32),
            pltpu.VMEM((CHUNK, D), jnp.float32),
            pltpu.VMEM((CHUNK, D), jnp.float32),
            pltpu.VMEM((CHUNK, D), jnp.float32),
            pltpu.VMEM((L,), jnp.float32),
            pltpu.SemaphoreType.DMA,
        ],
    )
    return f(head, rel, tail, negative_head, entity_table, relation_table)


def kernel(head, rel, tail, negative_head, negative_tail, entity_table,
           relation_table):
    del negative_tail
    partials = _transe_loss_partials(head, rel, tail, negative_head,
                                     entity_table, relation_table)
    return jnp.sum(partials)


# speedup vs baseline: 1.1754x; 1.0092x over previous
import jax
import jax.numpy as jnp
from jax import lax
from jax.experimental import pallas as pl
from jax.experimental.pallas import tpu as pltpu
from jax.experimental.pallas import tpu_sc as plsc

NC = 2
NS = 16
L = 16
NW = NC * NS
B = 16384
D = 128
DC = D // L
BPW = B // NW
CHUNK = 128
NCHUNK = BPW // CHUNK
EPS = 1e-6
GAMMA = 1.0


def _rsqrt(x):
    i = lax.bitcast_convert_type(x, jnp.int32)
    i = jnp.int32(0x5F3759DF) - lax.shift_right_logical(i, 1)
    y = lax.bitcast_convert_type(i, jnp.float32)
    for _ in range(2):
        y = y * (1.5 - 0.5 * x * y * y)
    return y


def _tec_body(head_r, rel_r, tail_r, nh_r, etab, rtab, out_r,
              idx_h, idx_r, idx_t, idx_n, rh, rr, rt, rn, loss_buf, sem):
    cid = lax.axis_index("c")
    sid = lax.axis_index("s")
    wid = sid * NC + cid
    base = wid * BPW

    pltpu.sync_copy(head_r.at[pl.ds(base, BPW)], idx_h)
    pltpu.sync_copy(rel_r.at[pl.ds(base, BPW)], idx_r)
    pltpu.sync_copy(tail_r.at[pl.ds(base, BPW)], idx_t)
    pltpu.sync_copy(nh_r.at[pl.ds(base, BPW)], idx_n)

    loss = jnp.float32(0.0)

    for ck in range(NCHUNK):
        o = ck * CHUNK
        cps = [
            pltpu.async_copy(etab.at[idx_h.at[pl.ds(o, CHUNK)]], rh, sem),
            pltpu.async_copy(rtab.at[idx_r.at[pl.ds(o, CHUNK)]], rr, sem),
            pltpu.async_copy(etab.at[idx_t.at[pl.ds(o, CHUNK)]], rt, sem),
            pltpu.async_copy(etab.at[idx_n.at[pl.ds(o, CHUNK)]], rn, sem),
        ]
        for cp in cps:
            cp.wait()

        @plsc.parallel_loop(0, CHUNK, unroll=2, carry=loss)
        def elem_body(e, loss):
            lh = [rh[e, pl.ds(k * L, L)] for k in range(DC)]
            lr = [rr[e, pl.ds(k * L, L)] for k in range(DC)]
            lt = [rt[e, pl.ds(k * L, L)] for k in range(DC)]
            ln = [rn[e, pl.ds(k * L, L)] for k in range(DC)]

            def sumsq(vs):
                acc = vs[0] * vs[0]
                for v in vs[1:]:
                    acc = acc + v * v
                return jnp.sum(acc)

            ah = _rsqrt(jnp.maximum(sumsq(lh), 1e-24))
            ar = _rsqrt(jnp.maximum(sumsq(lr), 1e-24))
            at = _rsqrt(jnp.maximum(sumsq(lt), 1e-24))
            an = _rsqrt(jnp.maximum(sumsq(ln), 1e-24))

            acc1 = None
            acc2 = None
            for k in range(DC):
                w = lr[k] * ar - lt[k] * at + EPS
                u = lh[k] * ah + w
                v = ln[k] * an + w
                acc1 = u * u if acc1 is None else acc1 + u * u
                acc2 = v * v if acc2 is None else acc2 + v * v
            s1 = jnp.maximum(jnp.sum(acc1), 1e-30)
            s2 = jnp.maximum(jnp.sum(acc2), 1e-30)
            d1 = s1 * _rsqrt(s1)
            d2 = s2 * _rsqrt(s2)
            return loss + jnp.maximum(GAMMA + d1 - d2, 0.0)

        loss = elem_body

    lane = lax.broadcasted_iota(jnp.int32, (L,), 0)
    loss_buf[...] = jnp.where(lane == 0, loss, 0.0)
    pltpu.sync_copy(loss_buf, out_r.at[wid])


@jax.jit
def _transe_loss_partials(head, rel, tail, negative_head, entity_table,
                          relation_table):
    mesh = plsc.VectorSubcoreMesh(
        core_axis_name="c", subcore_axis_name="s", num_cores=NC,
        num_subcores=NS)
    f = pl.kernel(
        _tec_body,
        out_type=jax.ShapeDtypeStruct((NW, L), jnp.float32),
        mesh=mesh,
        compiler_params=pltpu.CompilerParams(needs_layout_passes=False),
        scratch_types=[
            pltpu.VMEM((BPW,), jnp.int32),
            pltpu.VMEM((BPW,), jnp.int32),
            pltpu.VMEM((BPW,), jnp.int32),
            pltpu.VMEM((BPW,), jnp.int32),
            pltpu.VMEM((CHUNK, D), jnp.float32),
            pltpu.VMEM((CHUNK, D), jnp.float32),
            pltpu.VMEM((CHUNK, D), jnp.float32),
            pltpu.VMEM((CHUNK, D), jnp.float32),
            pltpu.VMEM((L,), jnp.float32),
            pltpu.SemaphoreType.DMA,
        ],
    )
    return f(head, rel, tail, negative_head, entity_table, relation_table)


def kernel(head, rel, tail, negative_head, negative_tail, entity_table,
           relation_table):
    del negative_tail
    partials = _transe_loss_partials(head, rel, tail, negative_head,
                                     entity_table, relation_table)
    return jnp.sum(partials)


# speedup vs baseline: 1.3102x; 1.1147x over previous
import jax
import jax.numpy as jnp
from jax import lax
from jax.experimental import pallas as pl
from jax.experimental.pallas import tpu as pltpu
from jax.experimental.pallas import tpu_sc as plsc

NC = 2
NS = 16
L = 16
NW = NC * NS
B = 16384
D = 128
DC = D // L
BPW = B // NW
CHUNK = 64
NCHUNK = BPW // CHUNK
EPS = 1e-6
GAMMA = 1.0


def _rsqrt(x):
    i = lax.bitcast_convert_type(x, jnp.int32)
    i = jnp.int32(0x5F3759DF) - lax.shift_right_logical(i, 1)
    y = lax.bitcast_convert_type(i, jnp.float32)
    for _ in range(2):
        y = y * (1.5 - 0.5 * x * y * y)
    return y


def _tec_body(head_r, rel_r, tail_r, nh_r, etab, rtab, out_r,
              idx_h, idx_r, idx_t, idx_n, rh, rr, rt, rn,
              rh2, rr2, rt2, rn2, loss_buf, sem, sem2):
    cid = lax.axis_index("c")
    sid = lax.axis_index("s")
    wid = sid * NC + cid
    base = wid * BPW

    pltpu.sync_copy(head_r.at[pl.ds(base, BPW)], idx_h)
    pltpu.sync_copy(rel_r.at[pl.ds(base, BPW)], idx_r)
    pltpu.sync_copy(tail_r.at[pl.ds(base, BPW)], idx_t)
    pltpu.sync_copy(nh_r.at[pl.ds(base, BPW)], idx_n)

    loss = jnp.float32(0.0)

    bufs = ((rh, rr, rt, rn), (rh2, rr2, rt2, rn2))
    sems = (sem, sem2)

    def start(ck):
        o = ck * CHUNK
        bh, br, bt, bn = bufs[ck % 2]
        s = sems[ck % 2]
        return [
            pltpu.async_copy(etab.at[idx_h.at[pl.ds(o, CHUNK)]], bh, s),
            pltpu.async_copy(rtab.at[idx_r.at[pl.ds(o, CHUNK)]], br, s),
            pltpu.async_copy(etab.at[idx_t.at[pl.ds(o, CHUNK)]], bt, s),
            pltpu.async_copy(etab.at[idx_n.at[pl.ds(o, CHUNK)]], bn, s),
        ]

    inflight = start(0)
    for ck in range(NCHUNK):
        nxt = start(ck + 1) if ck + 1 < NCHUNK else []
        for cp in inflight:
            cp.wait()
        inflight = nxt
        rhc, rrc, rtc, rnc = bufs[ck % 2]

        @plsc.parallel_loop(0, CHUNK, unroll=2, carry=loss)
        def elem_body(e, loss):
            lh = [rhc[e, pl.ds(k * L, L)] for k in range(DC)]
            lr = [rrc[e, pl.ds(k * L, L)] for k in range(DC)]
            lt = [rtc[e, pl.ds(k * L, L)] for k in range(DC)]
            ln = [rnc[e, pl.ds(k * L, L)] for k in range(DC)]

            def sumsq(vs):
                acc = vs[0] * vs[0]
                for v in vs[1:]:
                    acc = acc + v * v
                return jnp.sum(acc)

            ah = _rsqrt(jnp.maximum(sumsq(lh), 1e-24))
            ar = _rsqrt(jnp.maximum(sumsq(lr), 1e-24))
            at = _rsqrt(jnp.maximum(sumsq(lt), 1e-24))
            an = _rsqrt(jnp.maximum(sumsq(ln), 1e-24))

            acc1 = None
            acc2 = None
            for k in range(DC):
                w = lr[k] * ar - lt[k] * at + EPS
                u = lh[k] * ah + w
                v = ln[k] * an + w
                acc1 = u * u if acc1 is None else acc1 + u * u
                acc2 = v * v if acc2 is None else acc2 + v * v
            s1 = jnp.maximum(jnp.sum(acc1), 1e-30)
            s2 = jnp.maximum(jnp.sum(acc2), 1e-30)
            d1 = s1 * _rsqrt(s1)
            d2 = s2 * _rsqrt(s2)
            return loss + jnp.maximum(GAMMA + d1 - d2, 0.0)

        loss = elem_body

    lane = lax.broadcasted_iota(jnp.int32, (L,), 0)
    loss_buf[...] = jnp.where(lane == 0, loss, 0.0)
    pltpu.sync_copy(loss_buf, out_r.at[wid])


@jax.jit
def _transe_loss_partials(head, rel, tail, negative_head, entity_table,
                          relation_table):
    mesh = plsc.VectorSubcoreMesh(
        core_axis_name="c", subcore_axis_name="s", num_cores=NC,
        num_subcores=NS)
    f = pl.kernel(
        _tec_body,
        out_type=jax.ShapeDtypeStruct((NW, L), jnp.float32),
        mesh=mesh,
        compiler_params=pltpu.CompilerParams(needs_layout_passes=False),
        scratch_types=[
            pltpu.VMEM((BPW,), jnp.int32),
            pltpu.VMEM((BPW,), jnp.int32),
            pltpu.VMEM((BPW,), jnp.int32),
            pltpu.VMEM((BPW,), jnp.int32),
            pltpu.VMEM((CHUNK, D), jnp.float32),
            pltpu.VMEM((CHUNK, D), jnp.float32),
            pltpu.VMEM((CHUNK, D), jnp.float32),
            pltpu.VMEM((CHUNK, D), jnp.float32),
            pltpu.VMEM((CHUNK, D), jnp.float32),
            pltpu.VMEM((CHUNK, D), jnp.float32),
            pltpu.VMEM((CHUNK, D), jnp.float32),
            pltpu.VMEM((CHUNK, D), jnp.float32),
            pltpu.VMEM((L,), jnp.float32),
            pltpu.SemaphoreType.DMA,
            pltpu.SemaphoreType.DMA,
        ],
    )
    return f(head, rel, tail, negative_head, entity_table, relation_table)


def kernel(head, rel, tail, negative_head, negative_tail, entity_table,
           relation_table):
    del negative_tail
    partials = _transe_loss_partials(head, rel, tail, negative_head,
                                     entity_table, relation_table)
    return jnp.sum(partials)
